# Initial kernel scaffold; baseline (speedup 1.0000x reference)
import jax
import jax.numpy as jnp
from jax.experimental import pallas as pl


def kernel(x, weight, scale, threshold, expert_masks, router_w):
    raise NotImplementedError("write your pallas kernel here")


# speedup vs baseline: 2.2965x; 2.2965x over previous
import functools

import jax
import jax.numpy as jnp
from jax.experimental import pallas as pl

S, IN_F, OUT_F, E = 2048, 768, 768, 8
EPAD = 128
SBLK = 256


def _router_body(x_ref, rw_ref, w_ref, xw_ref, oh_ref, wr_ref):
    x = x_ref[...]
    logits = jax.lax.dot_general(
        x, rw_ref[...], (((1,), (1,)), ((), ())),
        preferred_element_type=jnp.float32)
    col = jax.lax.broadcasted_iota(jnp.int32, (S, EPAD), 1)
    lg = jnp.where(col < E, logits, jnp.float32(-1e30))
    m = jnp.max(lg, axis=1, keepdims=True)
    denom = jnp.sum(jnp.exp(lg - m), axis=1, keepdims=True)
    top1w = 1.0 / denom
    idx = jnp.min(jnp.where(lg >= m, col, EPAD), axis=1, keepdims=True)
    oh_ref[...] = (col == idx).astype(jnp.float32)
    xw_ref[...] = x * top1w
    wr_ref[...] = jnp.clip(jnp.round(w_ref[...] * 2.0), -1.0, 1.0)


def _accum_body(xw_ref, oh_ref, wr_ref, scale_ref, masks_ref, o_ref):
    acc = jnp.zeros((SBLK, OUT_F), jnp.float32)
    wr = wr_ref[...]
    xw = xw_ref[...]
    for e in range(E):
        wm = wr * masks_ref[e]
        ye = jax.lax.dot_general(
            xw, wm, (((1,), (1,)), ((), ())),
            preferred_element_type=jnp.float32)
        acc = acc + oh_ref[:, e:e + 1] * ye
    o_ref[...] = acc * scale_ref[...]


def _impl(x2, rw_pad, weight, scale_row, expert_masks, interpret=False):
    xw, oh, wr = pl.pallas_call(
        _router_body,
        out_shape=(
            jax.ShapeDtypeStruct((S, IN_F), jnp.float32),
            jax.ShapeDtypeStruct((S, EPAD), jnp.float32),
            jax.ShapeDtypeStruct((OUT_F, IN_F), jnp.float32),
        ),
        interpret=interpret,
    )(x2, rw_pad, weight)

    out = pl.pallas_call(
        _accum_body,
        grid=(S // SBLK,),
        in_specs=[
            pl.BlockSpec((SBLK, IN_F), lambda s: (s, 0)),
            pl.BlockSpec((SBLK, EPAD), lambda s: (s, 0)),
            pl.BlockSpec((OUT_F, IN_F), lambda s: (0, 0)),
            pl.BlockSpec((1, OUT_F), lambda s: (0, 0)),
            pl.BlockSpec((E, OUT_F, IN_F), lambda s: (0, 0, 0)),
        ],
        out_specs=pl.BlockSpec((SBLK, OUT_F), lambda s: (s, 0)),
        out_shape=jax.ShapeDtypeStruct((S, OUT_F), jnp.float32),
        interpret=interpret,
    )(xw, oh, wr, scale_row, expert_masks)
    return out


@jax.jit
def kernel(x, weight, scale, threshold, expert_masks, router_w):
    del threshold
    x2 = x.reshape(S, IN_F)
    rw_pad = jnp.zeros((EPAD, IN_F), jnp.float32).at[:E].set(router_w)
    scale_row = scale.reshape(1, OUT_F)
    out = _impl(x2, rw_pad, weight, scale_row, expert_masks)
    return out.reshape(1, S, OUT_F)
